# baseline (speedup 1.0000x reference)
import functools

import jax
import jax.numpy as jnp
from jax import lax
from jax.experimental import pallas as pl
from jax.experimental.pallas import tpu as pltpu
from jax.experimental.pallas import tpu_sc as plsc

_NC, _NS, _L = 2, 16, 16
_NW = _NC * _NS

_LOG1P_COEFS = (
    -0.01720806024968624,
    0.0817268118262291,
    -0.1887826770544052,
    0.31459054350852966,
    -0.49697792530059814,
    0.9997924566268921,
    3.50755203726294e-06,
)

_GATHER_DNUMS = lax.GatherDimensionNumbers(
    offset_dims=(), collapsed_slice_dims=(0,), start_index_map=(0,))


def _perm(x, idx):
    return lax.gather(x, idx[:, None], dimension_numbers=_GATHER_DNUMS,
                      slice_sizes=(1,),
                      mode=lax.GatherScatterMode.PROMISE_IN_BOUNDS)


def _softplus(x):
    t = jnp.exp(-jnp.abs(x))
    p = jnp.full(x.shape, _LOG1P_COEFS[0], jnp.float32)
    for c in _LOG1P_COEFS[1:]:
        p = p * t + c
    return jnp.maximum(x, 0.0) + p


def kernel(user_id, question_id, theta_table, a_table, b_table):
    B = user_id.shape[0]
    D = theta_table.shape[1]
    assert D == 2 * _L and B % (_NW * _L) == 0
    bw = B // _NW
    ng = bw // _L

    mesh = plsc.VectorSubcoreMesh(
        core_axis_name="c", subcore_axis_name="s",
        num_cores=_NC, num_subcores=_NS)

    @functools.partial(
        pl.kernel,
        out_type=jax.ShapeDtypeStruct((B,), jnp.float32),
        mesh=mesh,
        scratch_types=[
            pltpu.VMEM((bw,), jnp.int32),
            pltpu.VMEM((bw,), jnp.int32),
            pltpu.VMEM((bw, D), jnp.float32),
            pltpu.VMEM((bw, D), jnp.float32),
            pltpu.VMEM((bw,), jnp.float32),
            pltpu.VMEM((bw,), jnp.float32),
            pltpu.SemaphoreType.DMA,
            pltpu.SemaphoreType.DMA,
            pltpu.SemaphoreType.DMA,
        ],
        compiler_params=pltpu.CompilerParams(use_tc_tiling_on_sc=False),
    )
    def sc_kernel(uid_hbm, qid_hbm, th_hbm, a_hbm, b_hbm, out_hbm,
                  uid_v, qid_v, th_v, a_v, b_v, out_v, sem_th, sem_a, sem_b):
        wid = lax.axis_index("s") * _NC + lax.axis_index("c")
        base = wid * bw
        pltpu.sync_copy(uid_hbm.at[pl.ds(base, bw)], uid_v)
        pltpu.sync_copy(qid_hbm.at[pl.ds(base, bw)], qid_v)
        cp_th = pltpu.async_copy(th_hbm.at[uid_v], th_v, sem_th)
        cp_a = pltpu.async_copy(a_hbm.at[qid_v], a_v, sem_a)
        cp_b = pltpu.async_copy(b_hbm.at[qid_v], b_v, sem_b)
        cp_th.wait()
        cp_a.wait()
        cp_b.wait()

        lanes = lax.iota(jnp.int32, _L)

        def group(g, carry):
            res = jnp.zeros((_L,), jnp.float32)
            for i in range(_L):
                el = g * _L + i
                t0 = th_v[el, pl.ds(0, _L)]
                t1 = th_v[el, pl.ds(_L, _L)]
                a0 = a_v[el, pl.ds(0, _L)]
                a1 = a_v[el, pl.ds(_L, _L)]
                p = _softplus(a0) * t0 + _softplus(a1) * t1
                for s in (1, 2, 4, 8):
                    p = p + _perm(p, lanes ^ s)
                res = jnp.where(lanes == i, p, res)
            z = res - b_v[pl.ds(g * _L, _L)]
            out_v[pl.ds(g * _L, _L)] = 1.0 / (1.0 + jnp.exp(-z))
            return carry

        lax.fori_loop(0, ng, group, 0)
        pltpu.sync_copy(out_v, out_hbm.at[pl.ds(base, bw)])

    return sc_kernel(user_id, question_id, theta_table, a_table,
                     b_table.reshape(-1))
